# baseline (speedup 1.0000x reference)
import functools

import numpy as np

import jax
import jax.numpy as jnp
from jax import lax
from jax.experimental import pallas as pl
from jax.experimental.pallas import tpu as pltpu
from jax.experimental.pallas import tpu_sc as plsc

C = 384
H = 512
W = 512
N = H * W
NW = 32
R = 16
QS = N // NW
QD = N // R

_CONSTS = None


def _threefry2x32(k0, k1, x0, x1):
    u32 = np.uint32
    ks = [u32(k0), u32(k1), u32(k0) ^ u32(k1) ^ u32(0x1BD11BDA)]
    x = [x0.astype(u32).copy(), x1.astype(u32).copy()]

    def rounds(x, rots):
        for r in rots:
            x[0] = x[0] + x[1]
            x[1] = (x[1] << u32(r)) | (x[1] >> u32(32 - r))
            x[1] = x[0] ^ x[1]
        return x

    rot0, rot1 = [13, 15, 26, 6], [17, 29, 16, 24]
    x[0] = x[0] + ks[0]
    x[1] = x[1] + ks[1]
    x = rounds(x, rot0)
    x[0] = x[0] + ks[1]
    x[1] = x[1] + ks[2] + u32(1)
    x = rounds(x, rot1)
    x[0] = x[0] + ks[2]
    x[1] = x[1] + ks[0] + u32(2)
    x = rounds(x, rot0)
    x[0] = x[0] + ks[0]
    x[1] = x[1] + ks[1] + u32(3)
    x = rounds(x, rot1)
    x[0] = x[0] + ks[1]
    x[1] = x[1] + ks[2] + u32(4)
    x = rounds(x, rot0)
    x[0] = x[0] + ks[2]
    x[1] = x[1] + ks[0] + u32(5)
    return x[0], x[1]


def _np_permutation(seed, n):
    u32 = np.uint32
    key = (u32(np.int64(seed) >> 32), u32(np.int64(seed) & 0xFFFFFFFF))
    x = np.arange(n, dtype=np.int64)
    num_rounds = int(np.ceil(3 * np.log(max(1, n)) / np.log(2**32 - 1)))
    for _ in range(num_rounds):
        b1, b2 = _threefry2x32(
            key[0], key[1], np.zeros(2, u32), np.arange(2, dtype=u32)
        )
        key, subkey = (b1[0], b2[0]), (b1[1], b2[1])
        c1, c2 = np.zeros(n, u32), np.arange(n, dtype=u32)
        s1, s2 = _threefry2x32(subkey[0], subkey[1], c1, c2)
        x = x[np.argsort(s1 ^ s2, kind="stable")]
    return x


def _consts():
    global _CONSTS
    if _CONSTS is None:
        perm = _np_permutation(0, N)
        inv = np.empty(N, np.int64)
        inv[perm] = np.arange(N)
        w_of = np.arange(N) // QS
        r_of = inv // QD
        key = w_of * R + r_of
        counts = np.bincount(key, minlength=NW * R)
        starts = np.concatenate(([0], np.cumsum(counts)[:-1]))
        order = np.argsort(key, kind="stable")
        ranks = np.empty(N, np.int64)
        ranks[order] = np.arange(N) - np.repeat(starts, counts)
        pad = int(-(-counts.max() // 8) * 8)
        rho = (r_of * pad + ranks).astype(np.int32).reshape(NW, QS)
        sigma = (w_of[perm] * pad + ranks[perm]).astype(np.int32).reshape(R, QD)
        _CONSTS = (pad, rho, sigma)
    return _CONSTS


def _make_pass1(pad):
    mesh = plsc.VectorSubcoreMesh(core_axis_name="c", subcore_axis_name="s")

    @functools.partial(
        pl.kernel,
        mesh=mesh,
        compiler_params=pltpu.CompilerParams(needs_layout_passes=False),
        out_type=jax.ShapeDtypeStruct((C * NW * R * pad,), jnp.float32),
        scratch_types=[
            pltpu.VMEM((QS,), jnp.int32),
            pltpu.VMEM((QS,), jnp.float32),
            pltpu.VMEM((R * pad,), jnp.float32),
        ],
    )
    def pass1(x_hbm, rho_hbm, inter_hbm, rho_v, a_v, b_v):
        wid = lax.axis_index("s") * 2 + lax.axis_index("c")
        pltpu.sync_copy(rho_hbm.at[pl.ds(wid * QS, QS)], rho_v)

        def chan(c, carry):
            pltpu.sync_copy(x_hbm.at[pl.ds(c * N + wid * QS, QS)], a_v)

            def grp(j, carry2):
                idx = rho_v[pl.ds(j * 16, 16)]
                vals = a_v[pl.ds(j * 16, 16)]
                plsc.store_scatter(b_v, [idx], vals)
                return carry2

            lax.fori_loop(0, QS // 16, grp, 0, unroll=4)
            pltpu.sync_copy(
                b_v, inter_hbm.at[pl.ds((c * NW + wid) * (R * pad), R * pad)]
            )
            return carry

        lax.fori_loop(0, C, chan, 0)

    return pass1


def _make_pass2(pad):
    mesh = plsc.VectorSubcoreMesh(core_axis_name="c", subcore_axis_name="s")

    @functools.partial(
        pl.kernel,
        mesh=mesh,
        compiler_params=pltpu.CompilerParams(needs_layout_passes=False),
        out_type=jax.ShapeDtypeStruct((C * N,), jnp.float32),
        scratch_types=[
            pltpu.VMEM((QD,), jnp.int32),
            pltpu.VMEM((NW * pad,), jnp.float32),
            pltpu.VMEM((QD,), jnp.float32),
        ],
    )
    def pass2(inter_hbm, sigma_hbm, out_hbm, sig_v, rb_v, o_v):
        wid = lax.axis_index("s") * 2 + lax.axis_index("c")
        r = wid % R
        par = wid // R
        pltpu.sync_copy(sigma_hbm.at[pl.ds(r * QD, QD)], sig_v)

        def chan(i, carry):
            c = i * 2 + par
            for wp in range(NW):
                pltpu.sync_copy(
                    inter_hbm.at[
                        pl.ds((c * NW + wp) * (R * pad) + r * pad, pad)
                    ],
                    rb_v.at[pl.ds(wp * pad, pad)],
                )

            def grp(j, carry2):
                idx = sig_v[pl.ds(j * 16, 16)]
                o_v[pl.ds(j * 16, 16)] = plsc.load_gather(rb_v, [idx])
                return carry2

            lax.fori_loop(0, QD // 16, grp, 0, unroll=4)
            pltpu.sync_copy(o_v, out_hbm.at[pl.ds(c * N + r * QD, QD)])
            return carry

        lax.fori_loop(0, C // 2, chan, 0)

    return pass2


def kernel(x):
    pad, rho, sigma = _consts()
    x1d = x.reshape(C * N)
    rho_j = jnp.asarray(rho.reshape(-1))
    sigma_j = jnp.asarray(sigma.reshape(-1))
    inter = _make_pass1(pad)(x1d, rho_j)
    out = _make_pass2(pad)(inter, sigma_j)
    return out.reshape(C, H, W)


# speedup vs baseline: 2.1357x; 2.1357x over previous
import functools

import numpy as np

import jax
import jax.numpy as jnp
from jax import lax
from jax.experimental import pallas as pl
from jax.experimental.pallas import tpu as pltpu
from jax.experimental.pallas import tpu_sc as plsc

C = 384
H = 512
W = 512
N = H * W
NW = 32
R = 16
QS = N // NW
QD = N // R

_CONSTS = None


def _threefry2x32(k0, k1, x0, x1):
    u32 = np.uint32
    ks = [u32(k0), u32(k1), u32(k0) ^ u32(k1) ^ u32(0x1BD11BDA)]
    x = [x0.astype(u32).copy(), x1.astype(u32).copy()]

    def rounds(x, rots):
        for r in rots:
            x[0] = x[0] + x[1]
            x[1] = (x[1] << u32(r)) | (x[1] >> u32(32 - r))
            x[1] = x[0] ^ x[1]
        return x

    rot0, rot1 = [13, 15, 26, 6], [17, 29, 16, 24]
    x[0] = x[0] + ks[0]
    x[1] = x[1] + ks[1]
    x = rounds(x, rot0)
    x[0] = x[0] + ks[1]
    x[1] = x[1] + ks[2] + u32(1)
    x = rounds(x, rot1)
    x[0] = x[0] + ks[2]
    x[1] = x[1] + ks[0] + u32(2)
    x = rounds(x, rot0)
    x[0] = x[0] + ks[0]
    x[1] = x[1] + ks[1] + u32(3)
    x = rounds(x, rot1)
    x[0] = x[0] + ks[1]
    x[1] = x[1] + ks[2] + u32(4)
    x = rounds(x, rot0)
    x[0] = x[0] + ks[2]
    x[1] = x[1] + ks[0] + u32(5)
    return x[0], x[1]


def _np_permutation(seed, n):
    u32 = np.uint32
    key = (u32(np.int64(seed) >> 32), u32(np.int64(seed) & 0xFFFFFFFF))
    x = np.arange(n, dtype=np.int64)
    num_rounds = int(np.ceil(3 * np.log(max(1, n)) / np.log(2**32 - 1)))
    for _ in range(num_rounds):
        b1, b2 = _threefry2x32(
            key[0], key[1], np.zeros(2, u32), np.arange(2, dtype=u32)
        )
        key, subkey = (b1[0], b2[0]), (b1[1], b2[1])
        c1, c2 = np.zeros(n, u32), np.arange(n, dtype=u32)
        s1, s2 = _threefry2x32(subkey[0], subkey[1], c1, c2)
        x = x[np.argsort(s1 ^ s2, kind="stable")]
    return x


def _consts():
    global _CONSTS
    if _CONSTS is None:
        perm = _np_permutation(0, N)
        inv = np.empty(N, np.int64)
        inv[perm] = np.arange(N)
        w_of = np.arange(N) // QS
        r_of = inv // QD
        key = w_of * R + r_of
        counts = np.bincount(key, minlength=NW * R)
        starts = np.concatenate(([0], np.cumsum(counts)[:-1]))
        order = np.argsort(key, kind="stable")
        ranks = np.empty(N, np.int64)
        ranks[order] = np.arange(N) - np.repeat(starts, counts)
        pad = int(-(-counts.max() // 8) * 8)
        rho = (r_of * pad + ranks).astype(np.int32).reshape(NW, QS)
        sigma = (w_of[perm] * pad + ranks[perm]).astype(np.int32).reshape(R, QD)
        _CONSTS = (pad, rho, sigma)
    return _CONSTS


def _make_pass1(pad):
    mesh = plsc.VectorSubcoreMesh(core_axis_name="c", subcore_axis_name="s")

    @functools.partial(
        pl.kernel,
        mesh=mesh,
        compiler_params=pltpu.CompilerParams(needs_layout_passes=False),
        out_type=jax.ShapeDtypeStruct((C * NW * R * pad,), jnp.float32),
        scratch_types=[
            pltpu.VMEM((QS,), jnp.int32),
            pltpu.VMEM((QS,), jnp.float32),
            pltpu.VMEM((QS,), jnp.float32),
            pltpu.VMEM((R * pad,), jnp.float32),
            pltpu.VMEM((R * pad,), jnp.float32),
            pltpu.SemaphoreType.DMA,
            pltpu.SemaphoreType.DMA,
            pltpu.SemaphoreType.DMA,
            pltpu.SemaphoreType.DMA,
        ],
    )
    def pass1(x_hbm, rho_hbm, inter_hbm, rho_v, a0, a1, b0, b1,
              sa0, sa1, sb0, sb1):
        wid = lax.axis_index("s") * 2 + lax.axis_index("c")
        pltpu.sync_copy(rho_hbm.at[pl.ds(wid * QS, QS)], rho_v)

        def in_start(c, a_v, sem):
            cc = jnp.minimum(c, C - 1)
            pltpu.async_copy(x_hbm.at[pl.ds(cc * N + wid * QS, QS)], a_v, sem)

        def in_wait(a_v, sem):
            pltpu.make_async_copy(x_hbm.at[pl.ds(wid * QS, QS)], a_v, sem).wait()

        def out_start(c, b_v, sem):
            pltpu.async_copy(
                b_v, inter_hbm.at[pl.ds((c * NW + wid) * (R * pad), R * pad)],
                sem,
            )

        def out_wait(b_v, sem):
            pltpu.make_async_copy(
                b_v, inter_hbm.at[pl.ds(wid * (R * pad), R * pad)], sem
            ).wait()

        def shuffle(a_v, b_v):
            def grp(j, carry2):
                idx = rho_v[pl.ds(j * 16, 16)]
                vals = a_v[pl.ds(j * 16, 16)]
                plsc.store_scatter(b_v, [idx], vals)
                return carry2

            lax.fori_loop(0, QS // 16, grp, 0, unroll=8)

        in_start(0, a0, sa0)

        def step(i2, carry):
            for bsel, a_v, b_v, sa, sb in (
                (0, a0, b0, sa0, sb0),
                (1, a1, b1, sa1, sb1),
            ):
                c = i2 * 2 + bsel
                in_wait(a_v, sa)
                in_start(c + 1, (a1, a0)[bsel], (sa1, sa0)[bsel])

                @pl.when(i2 > 0)
                def _():
                    out_wait(b_v, sb)

                shuffle(a_v, b_v)
                out_start(c, b_v, sb)
            return carry

        lax.fori_loop(0, C // 2, step, 0)
        in_wait(a0, sa0)
        out_wait(b0, sb0)
        out_wait(b1, sb1)

    return pass1


def _make_pass2(pad):
    mesh = plsc.VectorSubcoreMesh(core_axis_name="c", subcore_axis_name="s")

    @functools.partial(
        pl.kernel,
        mesh=mesh,
        compiler_params=pltpu.CompilerParams(needs_layout_passes=False),
        out_type=jax.ShapeDtypeStruct((C * N,), jnp.float32),
        scratch_types=[
            pltpu.VMEM((QD,), jnp.int32),
            pltpu.VMEM((NW * pad,), jnp.float32),
            pltpu.VMEM((NW * pad,), jnp.float32),
            pltpu.VMEM((QD,), jnp.float32),
            pltpu.VMEM((QD,), jnp.float32),
            pltpu.SemaphoreType.DMA,
            pltpu.SemaphoreType.DMA,
            pltpu.SemaphoreType.DMA,
            pltpu.SemaphoreType.DMA,
        ],
    )
    def pass2(inter_hbm, sigma_hbm, out_hbm, sig_v, rb0, rb1, o0, o1,
              si0, si1, so0, so1):
        wid = lax.axis_index("s") * 2 + lax.axis_index("c")
        r = wid % R
        par = wid // R
        pltpu.sync_copy(sigma_hbm.at[pl.ds(r * QD, QD)], sig_v)
        npairs = C // 2

        def seg_start(i, rb_v, sem):
            c = jnp.minimum(i, npairs - 1) * 2 + par
            for wp in range(NW):
                pltpu.async_copy(
                    inter_hbm.at[
                        pl.ds((c * NW + wp) * (R * pad) + r * pad, pad)
                    ],
                    rb_v.at[pl.ds(wp * pad, pad)],
                    sem,
                )

        def seg_wait(rb_v, sem):
            for wp in range(NW):
                pltpu.make_async_copy(
                    inter_hbm.at[pl.ds(wp * (R * pad), pad)],
                    rb_v.at[pl.ds(wp * pad, pad)],
                    sem,
                ).wait()

        def out_start(i, o_v, sem):
            c = i * 2 + par
            pltpu.async_copy(o_v, out_hbm.at[pl.ds(c * N + r * QD, QD)], sem)

        def out_wait(o_v, sem):
            pltpu.make_async_copy(
                o_v, out_hbm.at[pl.ds(r * QD, QD)], sem
            ).wait()

        def gather(rb_v, o_v):
            def grp(j, carry2):
                idx = sig_v[pl.ds(j * 16, 16)]
                o_v[pl.ds(j * 16, 16)] = plsc.load_gather(rb_v, [idx])
                return carry2

            lax.fori_loop(0, QD // 16, grp, 0, unroll=8)

        seg_start(0, rb0, si0)

        def step(i2, carry):
            for bsel, rb_v, o_v, si, so in (
                (0, rb0, o0, si0, so0),
                (1, rb1, o1, si1, so1),
            ):
                i = i2 * 2 + bsel
                seg_wait(rb_v, si)
                seg_start(i + 1, (rb1, rb0)[bsel], (si1, si0)[bsel])

                @pl.when(i2 > 0)
                def _():
                    out_wait(o_v, so)

                gather(rb_v, o_v)
                out_start(i, o_v, so)
            return carry

        lax.fori_loop(0, npairs // 2, step, 0)
        seg_wait(rb0, si0)
        out_wait(o0, so0)
        out_wait(o1, so1)

    return pass2


def kernel(x):
    pad, rho, sigma = _consts()
    x1d = x.reshape(C * N)
    rho_j = jnp.asarray(rho.reshape(-1))
    sigma_j = jnp.asarray(sigma.reshape(-1))
    inter = _make_pass1(pad)(x1d, rho_j)
    out = _make_pass2(pad)(inter, sigma_j)
    return out.reshape(C, H, W)
